# baseline (speedup 1.0000x reference)
import functools

import jax
import jax.numpy as jnp
from jax import lax
from jax.experimental import pallas as pl
from jax.experimental.pallas import tpu as pltpu
from jax.experimental.pallas import tpu_sc as plsc

N = 320000
D = 128
NUM_SEG = 4096
NPAIR = 16
NBATCH = 16
SEG_P = NUM_SEG // NPAIR
CR = 128
NCH = N // CR
BUF = CR + 16
ROWS2D = N // 128
VOX = 0.0625


def _cluster_body(px_ref, py_ref, bt_ref, cl_ref, aux_ref):
    px = px_ref[...]
    py = py_ref[...]
    bt = bt_ref[...]
    sz = jnp.float32(VOX)
    x0 = jnp.min(px)
    x1 = jnp.max(px)
    y0 = jnp.min(py)
    y1 = jnp.max(py)
    b0 = jnp.min(bt)
    cx = jnp.floor((px - x0) / sz).astype(jnp.int32)
    cy = jnp.floor((py - y0) / sz).astype(jnp.int32)
    cb = bt - b0
    nvx = jnp.floor((x1 - x0) / sz).astype(jnp.int32) + 1
    nvy = jnp.floor((y1 - y0) / sz).astype(jnp.int32) + 1
    s = nvx * nvy
    cl_ref[...] = cx + cy * nvx + cb * s

    lane = lax.broadcasted_iota(jnp.int32, (8, 128), 1)
    row = lax.broadcasted_iota(jnp.int32, (8, 128), 0)
    off = jnp.zeros((8, 128), jnp.int32)
    for b in range(NBATCH):
        cnt = jnp.sum((cb == b).astype(jnp.int32))
        off = off + jnp.where(b < lane, cnt, 0)
    aux_ref[...] = jnp.where(row == 1, s, off)


def _make_sc_segmax():
    mesh = plsc.VectorSubcoreMesh(core_axis_name="c", subcore_axis_name="s")

    @functools.partial(
        pl.kernel,
        out_type=(jax.ShapeDtypeStruct((NUM_SEG, D), jnp.float32),
                  jax.ShapeDtypeStruct((NUM_SEG, D), jnp.float32)),
        mesh=mesh,
        scratch_types=[
            pltpu.VMEM((3, CR), jnp.int32),
            pltpu.VMEM((3, CR, D), jnp.float32),
            pltpu.VMEM((SEG_P + 8, D), jnp.float32),
            pltpu.VMEM((SEG_P + 8, D), jnp.float32),
            pltpu.VMEM((8, 128), jnp.int32),
            pltpu.SemaphoreType.DMA((3,)),
            pltpu.SemaphoreType.DMA((3,)),
            pltpu.SemaphoreType.DMA,
        ],
        compiler_params=pltpu.CompilerParams(needs_layout_passes=False),
    )
    def segmax(x_hbm, ids_hbm, aux_hbm, out_hbm, part_hbm, ids_v, slab_v,
               acc_v, accb_v, aux_v, sem_i, sem_x, sem_a):
        cid = lax.axis_index("c")
        sid = lax.axis_index("s")
        pair = sid // 2
        half = sid - 2 * pair
        p = cid * 8 + pair
        s0 = p * SEG_P

        neg = jnp.full((16,), -jnp.inf, jnp.float32)
        iota = lax.iota(jnp.int32, 16)

        pltpu.async_copy(aux_hbm, aux_v, sem_a).wait()

        def init_body(i, _):
            for cgrp in range(D // 16):
                acc_v[i, pl.ds(cgrp * 16, 16)] = neg
                accb_v[i, pl.ds(cgrp * 16, 16)] = neg
            return 0

        lax.fori_loop(0, SEG_P + 8, init_body, 0)

        s_vox = aux_v[1, pl.ds(0, 16)][0]
        cb_lo = jnp.minimum(s0 // s_vox, NBATCH)
        hi_idx = jnp.minimum((s0 + SEG_P - 1) // s_vox + 1, NBATCH)
        sel = jnp.where(iota == 0, cb_lo, jnp.where(iota == 1, hi_idx, 0))
        g = plsc.load_gather(aux_v, [jnp.zeros((16,), jnp.int32), sel])
        lo = g[0]
        hi = g[1]
        k_lo = lo // CR
        k_hi = (hi + CR - 1) // CR
        k_mid = (k_lo + k_hi) // 2
        my_lo = jnp.where(half == 0, k_lo, k_mid)
        my_hi = jnp.where(half == 0, k_mid, k_hi)

        def start_chunk(k):
            slot = lax.rem(k, 3)
            pltpu.make_async_copy(
                ids_hbm.at[pl.ds(k * CR, CR)], ids_v.at[slot],
                sem_i.at[slot],
            ).start()
            pltpu.make_async_copy(
                x_hbm.at[pl.ds(k * CR, CR)], slab_v.at[slot],
                sem_x.at[slot],
            ).start()

        @pl.when(my_lo < my_hi)
        def _():
            start_chunk(my_lo)

        @pl.when(my_lo + 1 < my_hi)
        def _():
            start_chunk(my_lo + 1)

        def chunk_body(k, _):
            kslot = lax.rem(k, 3)
            pltpu.make_async_copy(
                ids_hbm.at[pl.ds(k * CR, CR)], ids_v.at[kslot],
                sem_i.at[kslot],
            ).wait()

            @pl.when(k + 2 < my_hi)
            def _():
                start_chunk(k + 2)

            pltpu.make_async_copy(
                x_hbm.at[pl.ds(k * CR, CR)], slab_v.at[kslot],
                sem_x.at[kslot],
            ).wait()

            def rmw_body(j, _):
                ids16 = ids_v[kslot, pl.ds(j * 16, 16)]
                off = ids16 - s0
                mask = (off >= 0) & (off < SEG_P)
                so16 = jnp.where(mask, off, SEG_P)
                sos = [so16[l] for l in range(16)]
                for l in range(16):
                    so = sos[l]
                    acc = acc_v if l % 2 == 0 else accb_v
                    accs = [acc[so, pl.ds(c * 16, 16)]
                            for c in range(D // 16)]
                    rows = [slab_v[kslot, j * 16 + l, pl.ds(c * 16, 16)]
                            for c in range(D // 16)]
                    for c in range(D // 16):
                        acc[so, pl.ds(c * 16, 16)] = jnp.maximum(
                            accs[c], rows[c]
                        )
                return 0

            lax.fori_loop(0, CR // 16, rmw_body, 0)
            return 0

        lax.fori_loop(my_lo, my_hi, chunk_body, 0)

        def fold_body(i, _):
            for cgrp in range(D // 16):
                cs = pl.ds(cgrp * 16, 16)
                acc_v[i, cs] = jnp.maximum(acc_v[i, cs], accb_v[i, cs])
            return 0

        lax.fori_loop(0, SEG_P, fold_body, 0)

        @pl.when(half == 1)
        def _():
            pltpu.sync_copy(acc_v.at[pl.ds(0, SEG_P)],
                            part_hbm.at[pl.ds(s0, SEG_P)])

        plsc.subcore_barrier()

        @pl.when(half == 0)
        def _():
            pltpu.sync_copy(part_hbm.at[pl.ds(s0, SEG_P)],
                            accb_v.at[pl.ds(0, SEG_P)])

            def fin_body(i, _):
                for cgrp in range(D // 16):
                    cs = pl.ds(cgrp * 16, 16)
                    v = jnp.maximum(acc_v[i, cs], accb_v[i, cs])
                    acc_v[i, cs] = jnp.where(v == -jnp.inf,
                                             jnp.float32(0.0), v)
                return 0

            lax.fori_loop(0, SEG_P, fin_body, 0)
            pltpu.sync_copy(acc_v.at[pl.ds(0, SEG_P)],
                            out_hbm.at[pl.ds(s0, SEG_P)])

    return segmax


_sc_segmax = _make_sc_segmax()


def kernel(x, pos, batch):
    px = pos[:, 0].reshape(ROWS2D, 128)
    py = pos[:, 1].reshape(ROWS2D, 128)
    bt = batch.reshape(ROWS2D, 128)
    cluster, aux = pl.pallas_call(
        _cluster_body,
        out_shape=(
            jax.ShapeDtypeStruct((ROWS2D, 128), jnp.int32),
            jax.ShapeDtypeStruct((8, 128), jnp.int32),
        ),
    )(px, py, bt)
    return _sc_segmax(x, cluster.reshape(N), aux)[0]


# speedup vs baseline: 1.0083x; 1.0083x over previous
import functools

import jax
import jax.numpy as jnp
from jax import lax
from jax.experimental import pallas as pl
from jax.experimental.pallas import tpu as pltpu
from jax.experimental.pallas import tpu_sc as plsc

N = 320000
D = 128
NUM_SEG = 4096
NPAIR = 16
NBATCH = 16
SEG_P = NUM_SEG // NPAIR
CR = 128
ROWS2D = N // 128
VOX = 0.0625


def _cluster_body(px_ref, py_ref, bt_ref, cl_ref, aux_ref):
    px = px_ref[...]
    py = py_ref[...]
    bt = bt_ref[...]
    sz = jnp.float32(VOX)
    x0 = jnp.min(px)
    x1 = jnp.max(px)
    y0 = jnp.min(py)
    y1 = jnp.max(py)
    b0 = jnp.min(bt)
    cx = jnp.floor((px - x0) / sz).astype(jnp.int32)
    cy = jnp.floor((py - y0) / sz).astype(jnp.int32)
    cb = bt - b0
    nvx = jnp.floor((x1 - x0) / sz).astype(jnp.int32) + 1
    nvy = jnp.floor((y1 - y0) / sz).astype(jnp.int32) + 1
    s = nvx * nvy
    cl_ref[...] = cx + cy * nvx + cb * s

    lane = lax.broadcasted_iota(jnp.int32, (8, 128), 1)
    row = lax.broadcasted_iota(jnp.int32, (8, 128), 0)
    off = jnp.zeros((8, 128), jnp.int32)
    for b in range(NBATCH):
        cnt = jnp.sum((cb == b).astype(jnp.int32))
        off = off + jnp.where(b < lane, cnt, 0)
    aux_ref[...] = jnp.where(row == 1, s, off)


def _make_sc_segmax():
    mesh = plsc.VectorSubcoreMesh(core_axis_name="c", subcore_axis_name="s")

    @functools.partial(
        pl.kernel,
        out_type=jax.ShapeDtypeStruct((NUM_SEG, D), jnp.float32),
        mesh=mesh,
        scratch_types=[
            pltpu.VMEM((2, CR), jnp.int32),
            pltpu.VMEM((2, CR, D), jnp.float32),
            pltpu.VMEM((SEG_P + 8, D), jnp.float32),
            pltpu.VMEM((SEG_P + 8, D), jnp.float32),
            pltpu.VMEM((8, 128), jnp.int32),
            pltpu.VMEM_SHARED((8, SEG_P, D), jnp.float32),
            pltpu.SemaphoreType.DMA((2,)),
            pltpu.SemaphoreType.DMA((2,)),
            pltpu.SemaphoreType.DMA,
        ],
        compiler_params=pltpu.CompilerParams(needs_layout_passes=False),
    )
    def segmax(x_hbm, ids_hbm, aux_hbm, out_hbm, ids_v, slab_v,
               acc_v, accb_v, aux_v, shr_v, sem_i, sem_x, sem_a):
        cid = lax.axis_index("c")
        sid = lax.axis_index("s")
        pair = sid // 2
        half = sid - 2 * pair
        p = cid * 8 + pair
        s0 = p * SEG_P

        neg = jnp.full((16,), -jnp.inf, jnp.float32)
        iota = lax.iota(jnp.int32, 16)

        pltpu.async_copy(aux_hbm, aux_v, sem_a).wait()

        def init_body(i, _):
            for cgrp in range(D // 16):
                acc_v[i, pl.ds(cgrp * 16, 16)] = neg
                accb_v[i, pl.ds(cgrp * 16, 16)] = neg
            return 0

        lax.fori_loop(0, SEG_P + 8, init_body, 0)

        s_vox = aux_v[1, pl.ds(0, 16)][0]
        cb_lo = jnp.minimum(s0 // s_vox, NBATCH)
        hi_idx = jnp.minimum((s0 + SEG_P - 1) // s_vox + 1, NBATCH)
        sel = jnp.where(iota == 0, cb_lo, jnp.where(iota == 1, hi_idx, 0))
        g = plsc.load_gather(aux_v, [jnp.zeros((16,), jnp.int32), sel])
        lo = g[0]
        hi = g[1]
        k_lo = lo // CR
        k_hi = (hi + CR - 1) // CR
        k_mid = (k_lo + k_hi) // 2
        my_lo = jnp.where(half == 0, k_lo, k_mid)
        my_hi = jnp.where(half == 0, k_mid, k_hi)

        def start_chunk(k):
            slot = lax.rem(k, 2)
            pltpu.make_async_copy(
                ids_hbm.at[pl.ds(k * CR, CR)], ids_v.at[slot],
                sem_i.at[slot],
            ).start()
            pltpu.make_async_copy(
                x_hbm.at[pl.ds(k * CR, CR)], slab_v.at[slot],
                sem_x.at[slot],
            ).start()

        @pl.when(my_lo < my_hi)
        def _():
            start_chunk(my_lo)

        def chunk_body(k, _):
            kslot = lax.rem(k, 2)
            pltpu.make_async_copy(
                ids_hbm.at[pl.ds(k * CR, CR)], ids_v.at[kslot],
                sem_i.at[kslot],
            ).wait()

            @pl.when(k + 1 < my_hi)
            def _():
                start_chunk(k + 1)

            pltpu.make_async_copy(
                x_hbm.at[pl.ds(k * CR, CR)], slab_v.at[kslot],
                sem_x.at[kslot],
            ).wait()

            def rmw_body(j, _):
                ids16 = ids_v[kslot, pl.ds(j * 16, 16)]
                off = ids16 - s0
                mask = (off >= 0) & (off < SEG_P)
                so16 = jnp.where(mask, off, SEG_P)
                sos = [so16[l] for l in range(16)]
                for l in range(16):
                    so = sos[l]
                    acc = acc_v if l % 2 == 0 else accb_v
                    accs = [acc[so, pl.ds(c * 16, 16)]
                            for c in range(D // 16)]
                    rows = [slab_v[kslot, j * 16 + l, pl.ds(c * 16, 16)]
                            for c in range(D // 16)]
                    for c in range(D // 16):
                        acc[so, pl.ds(c * 16, 16)] = jnp.maximum(
                            accs[c], rows[c]
                        )
                return 0

            lax.fori_loop(0, CR // 16, rmw_body, 0)
            return 0

        lax.fori_loop(my_lo, my_hi, chunk_body, 0)

        def fold_body(i, _):
            for cgrp in range(D // 16):
                cs = pl.ds(cgrp * 16, 16)
                acc_v[i, cs] = jnp.maximum(acc_v[i, cs], accb_v[i, cs])
            return 0

        lax.fori_loop(0, SEG_P, fold_body, 0)

        @pl.when(half == 1)
        def _():
            pltpu.sync_copy(acc_v.at[pl.ds(0, SEG_P)], shr_v.at[pair])

        plsc.subcore_barrier()

        @pl.when(half == 0)
        def _():
            pltpu.sync_copy(shr_v.at[pair], accb_v.at[pl.ds(0, SEG_P)])

            def fin_body(i, _):
                for cgrp in range(D // 16):
                    cs = pl.ds(cgrp * 16, 16)
                    v = jnp.maximum(acc_v[i, cs], accb_v[i, cs])
                    acc_v[i, cs] = jnp.where(v == -jnp.inf,
                                             jnp.float32(0.0), v)
                return 0

            lax.fori_loop(0, SEG_P, fin_body, 0)
            pltpu.sync_copy(acc_v.at[pl.ds(0, SEG_P)],
                            out_hbm.at[pl.ds(s0, SEG_P)])

    return segmax


_sc_segmax = _make_sc_segmax()


def kernel(x, pos, batch):
    px = pos[:, 0].reshape(ROWS2D, 128)
    py = pos[:, 1].reshape(ROWS2D, 128)
    bt = batch.reshape(ROWS2D, 128)
    cluster, aux = pl.pallas_call(
        _cluster_body,
        out_shape=(
            jax.ShapeDtypeStruct((ROWS2D, 128), jnp.int32),
            jax.ShapeDtypeStruct((8, 128), jnp.int32),
        ),
    )(px, py, bt)
    return _sc_segmax(x, cluster.reshape(N), aux)
